# baseline (speedup 1.0000x reference)
import jax
import jax.numpy as jnp
from jax import lax
from jax.experimental import pallas as pl
from jax.experimental.pallas import tpu as pltpu
from jax.experimental.pallas import tpu_sc as plsc

N = 10000
D = 128
E = 320000
NEG = 0.01
L = 16
NW = 32
R = 5
CHUNK = 2560
NCHUNK = E // CHUNK
VPC = CHUNK // L
UNROLL = 8


def _sc_body(f_hbm, src_hbm, dst_hbm, out_hbm, f_buf, acc0, acc1, acc2,
             acc3, acc4, probe, src_bufs, dst_bufs, sem0, sem1):
    nc = 2
    wid = lax.axis_index("s") * nc + lax.axis_index("c")
    accs = [acc0, acc1, acc2, acc3, acc4]

    pltpu.sync_copy(f_hbm.at[wid], f_buf)
    for r in range(R - 1):
        pltpu.sync_copy(f_hbm.at[wid, r], accs[r])
    zeros = jnp.zeros((L,), jnp.float32)

    def zero_row4(j, carry):
        acc4[pl.ds(j * L, L)] = zeros
        return carry

    lax.fori_loop(0, N // L, zero_row4, 0)

    lane = lax.iota(jnp.int32, L)
    row_ids = [jnp.full((L,), r, jnp.int32) for r in range(R)]
    tags = [lane + u * L for u in range(UNROLL)]

    def issue(ci, sbuf, dbuf, sem):
        e0 = ci * CHUNK
        pltpu.async_copy(src_hbm.at[pl.ds(e0, CHUNK)], sbuf, sem)
        pltpu.async_copy(dst_hbm.at[pl.ds(e0, CHUNK)], dbuf, sem)

    def drain(sbuf, dbuf, sem):
        pltpu.make_async_copy(src_hbm.at[pl.ds(0, CHUNK)], sbuf, sem).wait()
        pltpu.make_async_copy(dst_hbm.at[pl.ds(0, CHUNK)], dbuf, sem).wait()

    def retry_scatter_max(d_idx, vals, active0):
        def w_cond(active):
            return jnp.any(active)

        def w_body(active):
            plsc.store_scatter(probe, [d_idx], lane, mask=active)
            back = plsc.load_gather(probe, [d_idx])
            won = jnp.logical_and(back == lane, active)
            for r in range(R):
                cur = plsc.load_gather(accs[r], [d_idx])
                plsc.store_scatter(accs[r], [d_idx],
                                   jnp.maximum(cur, vals[r]), mask=won)
            return jnp.logical_and(active, jnp.logical_not(won))

        lax.while_loop(w_cond, w_body, active0)

    def compute_chunk(p):
        def vec_body(vi, carry):
            d_idxs, valss = [], []
            for u in range(UNROLL):
                v = vi * UNROLL + u
                s_idx = src_bufs[p, pl.ds(v * L, L)]
                d_idx = dst_bufs[p, pl.ds(v * L, L)]
                vals = [plsc.load_gather(f_buf, [row_ids[r], s_idx])
                        for r in range(R - 1)]
                vals.append(
                    plsc.load_gather(f_buf, [row_ids[R - 1], s_idx])
                    - plsc.load_gather(f_buf, [row_ids[R - 1], d_idx]))
                d_idxs.append(d_idx)
                valss.append(vals)
            for u in range(UNROLL):
                plsc.store_scatter(probe, [d_idxs[u]], tags[u])
            backs = [plsc.load_gather(probe, [d_idxs[u]])
                     for u in range(UNROLL)]
            wons = [backs[u] == tags[u] for u in range(UNROLL)]
            curs = [[plsc.load_gather(accs[r], [d_idxs[u]])
                     for r in range(R)] for u in range(UNROLL)]
            for u in range(UNROLL):
                for r in range(R):
                    plsc.store_scatter(accs[r], [d_idxs[u]],
                                       jnp.maximum(curs[u][r], valss[u][r]),
                                       mask=wons[u])
            all_won = wons[0]
            for u in range(1, UNROLL):
                all_won = jnp.logical_and(all_won, wons[u])

            @pl.when(jnp.logical_not(jnp.all(all_won)))
            def _():
                for u in range(UNROLL):
                    retry_scatter_max(d_idxs[u], valss[u],
                                      jnp.logical_not(wons[u]))

            return carry

        lax.fori_loop(0, VPC // UNROLL, vec_body, 0)

    issue(0, src_bufs.at[0], dst_bufs.at[0], sem0)

    def pair_body(k, carry):
        ci0 = 2 * k
        drain(src_bufs.at[0], dst_bufs.at[0], sem0)
        issue(ci0 + 1, src_bufs.at[1], dst_bufs.at[1], sem1)
        compute_chunk(0)
        drain(src_bufs.at[1], dst_bufs.at[1], sem1)
        issue(ci0 + 2, src_bufs.at[0], dst_bufs.at[0], sem0)
        compute_chunk(1)
        return carry

    lax.fori_loop(0, NCHUNK // 2, pair_body, 0)
    drain(src_bufs.at[0], dst_bufs.at[0], sem0)
    compute_chunk(0)

    def act_body(j, carry):
        for r in range(R):
            v = accs[r][pl.ds(j * L, L)]
            accs[r][pl.ds(j * L, L)] = jnp.where(v >= 0, v, NEG * v)
        return carry

    lax.fori_loop(0, N // L, act_body, 0)
    for r in range(R):
        pltpu.sync_copy(accs[r], out_hbm.at[wid, r])


def kernel(x, pos, edge_index):
    x_part = x.T.reshape(NW, 4, N)
    pos_part = jnp.concatenate(
        [pos.T, jnp.zeros((NW - 3, N), jnp.float32)], axis=0
    ).reshape(NW, 1, N)
    f_pad = jnp.concatenate([x_part, pos_part], axis=1)

    mesh = plsc.VectorSubcoreMesh(core_axis_name="c", subcore_axis_name="s",
                                  num_cores=2, num_subcores=16)
    out_pad = pl.kernel(
        _sc_body,
        out_type=jax.ShapeDtypeStruct((NW, R, N), jnp.float32),
        mesh=mesh,
        compiler_params=pltpu.CompilerParams(use_tc_tiling_on_sc=False,
                                             needs_layout_passes=False),
        scratch_types=[
            pltpu.VMEM((R, N), jnp.float32),
            pltpu.VMEM((N,), jnp.float32),
            pltpu.VMEM((N,), jnp.float32),
            pltpu.VMEM((N,), jnp.float32),
            pltpu.VMEM((N,), jnp.float32),
            pltpu.VMEM((N,), jnp.float32),
            pltpu.VMEM((N,), jnp.int32),
            pltpu.VMEM((2, CHUNK), jnp.int32),
            pltpu.VMEM((2, CHUNK), jnp.int32),
            pltpu.SemaphoreType.DMA,
            pltpu.SemaphoreType.DMA,
        ],
    )(f_pad, edge_index[0], edge_index[1])

    out_rows = out_pad.reshape(NW * R, N)
    row_map = [5 * (c // 4) + c % 4 for c in range(D)] + [5 * p + 4
                                                         for p in range(3)]
    return out_rows[jnp.array(row_map), :].T


# speedup vs baseline: 1.9205x; 1.9205x over previous
import jax
import jax.numpy as jnp
from jax import lax
from jax.experimental import pallas as pl
from jax.experimental.pallas import tpu as pltpu
from jax.experimental.pallas import tpu_sc as plsc

N = 10000
D = 128
E = 320000
NEG = 0.01
L = 16
NW = 32
R = 5
CHUNK = 2560
NCHUNK = E // CHUNK
VPC = CHUNK // L
UNROLL = 5


def _sc_body(f_hbm, src_hbm, dst_hbm, out_hbm, f_buf, acc0, acc1, acc2,
             acc3, acc4, probe, src_bufs, dst_bufs, sem0, sem1):
    nc = 2
    wid = lax.axis_index("s") * nc + lax.axis_index("c")
    accs = [acc0, acc1, acc2, acc3, acc4]

    pltpu.sync_copy(f_hbm.at[wid], f_buf)
    for r in range(R - 1):
        pltpu.sync_copy(f_hbm.at[wid, r], accs[r])
    zeros = jnp.zeros((L,), jnp.float32)

    def zero_row4(j, carry):
        acc4[pl.ds(j * L, L)] = zeros
        return carry

    lax.fori_loop(0, N // L, zero_row4, 0)

    lane = lax.iota(jnp.int32, L)
    row_ids = [jnp.full((L,), r, jnp.int32) for r in range(R)]
    tags = [lane + u * L for u in range(UNROLL)]

    def issue(ci, sbuf, dbuf, sem):
        e0 = ci * CHUNK
        pltpu.async_copy(src_hbm.at[pl.ds(e0, CHUNK)], sbuf, sem)
        pltpu.async_copy(dst_hbm.at[pl.ds(e0, CHUNK)], dbuf, sem)

    def drain(sbuf, dbuf, sem):
        pltpu.make_async_copy(src_hbm.at[pl.ds(0, CHUNK)], sbuf, sem).wait()
        pltpu.make_async_copy(dst_hbm.at[pl.ds(0, CHUNK)], dbuf, sem).wait()

    def retry_scatter_max(d_idx, vals, active0):
        def w_cond(active):
            return jnp.any(active)

        def w_body(active):
            plsc.store_scatter(probe, [d_idx], lane, mask=active)
            back = plsc.load_gather(probe, [d_idx])
            won = jnp.logical_and(back == lane, active)
            for r in range(R):
                cur = plsc.load_gather(accs[r], [d_idx])
                plsc.store_scatter(accs[r], [d_idx],
                                   jnp.maximum(cur, vals[r]), mask=won)
            return jnp.logical_and(active, jnp.logical_not(won))

        lax.while_loop(w_cond, w_body, active0)

    def compute_chunk(p):
        def vec_body(vi, carry):
            d_idxs, valss = [], []
            for u in range(UNROLL):
                v = vi * UNROLL + u
                s_idx = src_bufs[p, pl.ds(v * L, L)]
                d_idx = dst_bufs[p, pl.ds(v * L, L)]
                vals = [plsc.load_gather(f_buf, [row_ids[r], s_idx])
                        for r in range(R - 1)]
                vals.append(
                    plsc.load_gather(f_buf, [row_ids[R - 1], s_idx])
                    - plsc.load_gather(f_buf, [row_ids[R - 1], d_idx]))
                d_idxs.append(d_idx)
                valss.append(vals)
            for u in range(UNROLL):
                plsc.store_scatter(probe, [d_idxs[u]], tags[u])
            backs = [plsc.load_gather(probe, [d_idxs[u]])
                     for u in range(UNROLL)]
            wons = [backs[u] == tags[u] for u in range(UNROLL)]
            curs = [[plsc.load_gather(accs[r], [d_idxs[u]])
                     for r in range(R)] for u in range(UNROLL)]
            for u in range(UNROLL):
                for r in range(R):
                    plsc.store_scatter(accs[r], [d_idxs[u]],
                                       jnp.maximum(curs[u][r], valss[u][r]),
                                       mask=wons[u])
            all_won = wons[0]
            for u in range(1, UNROLL):
                all_won = jnp.logical_and(all_won, wons[u])

            @pl.when(jnp.logical_not(jnp.all(all_won)))
            def _():
                for u in range(UNROLL):
                    retry_scatter_max(d_idxs[u], valss[u],
                                      jnp.logical_not(wons[u]))

            return carry

        lax.fori_loop(0, VPC // UNROLL, vec_body, 0)

    issue(0, src_bufs.at[0], dst_bufs.at[0], sem0)

    def pair_body(k, carry):
        ci0 = 2 * k
        drain(src_bufs.at[0], dst_bufs.at[0], sem0)
        issue(ci0 + 1, src_bufs.at[1], dst_bufs.at[1], sem1)
        compute_chunk(0)
        drain(src_bufs.at[1], dst_bufs.at[1], sem1)
        issue(ci0 + 2, src_bufs.at[0], dst_bufs.at[0], sem0)
        compute_chunk(1)
        return carry

    lax.fori_loop(0, NCHUNK // 2, pair_body, 0)
    drain(src_bufs.at[0], dst_bufs.at[0], sem0)
    compute_chunk(0)

    def act_body(j, carry):
        for r in range(R):
            v = accs[r][pl.ds(j * L, L)]
            accs[r][pl.ds(j * L, L)] = jnp.where(v >= 0, v, NEG * v)
        return carry

    lax.fori_loop(0, N // L, act_body, 0)
    for r in range(R):
        pltpu.sync_copy(accs[r], out_hbm.at[wid, r])


def kernel(x, pos, edge_index):
    x_part = x.T.reshape(NW, 4, N)
    pos_part = jnp.concatenate(
        [pos.T, jnp.zeros((NW - 3, N), jnp.float32)], axis=0
    ).reshape(NW, 1, N)
    f_pad = jnp.concatenate([x_part, pos_part], axis=1)

    mesh = plsc.VectorSubcoreMesh(core_axis_name="c", subcore_axis_name="s",
                                  num_cores=2, num_subcores=16)
    out_pad = pl.kernel(
        _sc_body,
        out_type=jax.ShapeDtypeStruct((NW, R, N), jnp.float32),
        mesh=mesh,
        compiler_params=pltpu.CompilerParams(use_tc_tiling_on_sc=False,
                                             needs_layout_passes=False),
        scratch_types=[
            pltpu.VMEM((R, N), jnp.float32),
            pltpu.VMEM((N,), jnp.float32),
            pltpu.VMEM((N,), jnp.float32),
            pltpu.VMEM((N,), jnp.float32),
            pltpu.VMEM((N,), jnp.float32),
            pltpu.VMEM((N,), jnp.float32),
            pltpu.VMEM((N,), jnp.int32),
            pltpu.VMEM((2, CHUNK), jnp.int32),
            pltpu.VMEM((2, CHUNK), jnp.int32),
            pltpu.SemaphoreType.DMA,
            pltpu.SemaphoreType.DMA,
        ],
    )(f_pad, edge_index[0], edge_index[1])

    out_rows = out_pad.reshape(NW * R, N)
    row_map = [5 * (c // 4) + c % 4 for c in range(D)] + [5 * p + 4
                                                         for p in range(3)]
    return out_rows[jnp.array(row_map), :].T


# speedup vs baseline: 2.1200x; 1.1039x over previous
import jax
import jax.numpy as jnp
from jax import lax
from jax.experimental import pallas as pl
from jax.experimental.pallas import tpu as pltpu
from jax.experimental.pallas import tpu_sc as plsc

N = 10000
D = 128
E = 320000
NEG = 0.01
L = 16
NW = 32
R = 5
CHUNK = 2560
NCHUNK = E // CHUNK
VPC = CHUNK // L
UNROLL = 4


def _sc_body(f_hbm, edge_hbm, out_hbm, f_buf, acc0, acc1, acc2,
             acc3, acc4, probe, src_bufs, dst_bufs, sem0, sem1):
    nc = 2
    wid = lax.axis_index("s") * nc + lax.axis_index("c")
    accs = [acc0, acc1, acc2, acc3, acc4]

    pltpu.sync_copy(f_hbm.at[wid], f_buf)
    for r in range(R - 1):
        pltpu.sync_copy(f_hbm.at[wid, r], accs[r])
    zeros = jnp.zeros((L,), jnp.float32)

    def zero_row4(j, carry):
        acc4[pl.ds(j * L, L)] = zeros
        return carry

    lax.fori_loop(0, N // L, zero_row4, 0)

    lane = lax.iota(jnp.int32, L)
    row_ids = [jnp.full((L,), r, jnp.int32) for r in range(R)]
    tags = [lane + u * L for u in range(UNROLL)]

    def issue(ci, sbuf, dbuf, sem):
        e0 = ci * CHUNK
        pltpu.async_copy(edge_hbm.at[0, pl.ds(e0, CHUNK)], sbuf, sem)
        pltpu.async_copy(edge_hbm.at[1, pl.ds(e0, CHUNK)], dbuf, sem)

    def drain(sbuf, dbuf, sem):
        pltpu.make_async_copy(edge_hbm.at[0, pl.ds(0, CHUNK)], sbuf,
                              sem).wait()
        pltpu.make_async_copy(edge_hbm.at[1, pl.ds(0, CHUNK)], dbuf,
                              sem).wait()

    def retry_scatter_max(d_idx, vals, active0):
        def w_cond(active):
            return jnp.any(active)

        def w_body(active):
            plsc.store_scatter(probe, [d_idx], lane, mask=active)
            back = plsc.load_gather(probe, [d_idx])
            won = jnp.logical_and(back == lane, active)
            for r in range(R):
                cur = plsc.load_gather(accs[r], [d_idx])
                plsc.store_scatter(accs[r], [d_idx],
                                   jnp.maximum(cur, vals[r]), mask=won)
            return jnp.logical_and(active, jnp.logical_not(won))

        lax.while_loop(w_cond, w_body, active0)

    def compute_chunk(p):
        def vec_body(vi, carry):
            d_idxs, valss = [], []
            for u in range(UNROLL):
                v = vi * UNROLL + u
                s_idx = src_bufs[p, pl.ds(v * L, L)]
                d_idx = dst_bufs[p, pl.ds(v * L, L)]
                vals = [plsc.load_gather(f_buf, [row_ids[r], s_idx])
                        for r in range(R - 1)]
                vals.append(
                    plsc.load_gather(f_buf, [row_ids[R - 1], s_idx])
                    - plsc.load_gather(f_buf, [row_ids[R - 1], d_idx]))
                d_idxs.append(d_idx)
                valss.append(vals)
            for u in range(UNROLL):
                plsc.store_scatter(probe, [d_idxs[u]], tags[u])
            backs = [plsc.load_gather(probe, [d_idxs[u]])
                     for u in range(UNROLL)]
            wons = [backs[u] == tags[u] for u in range(UNROLL)]
            curs = [[plsc.load_gather(accs[r], [d_idxs[u]])
                     for r in range(R)] for u in range(UNROLL)]
            for u in range(UNROLL):
                for r in range(R):
                    plsc.store_scatter(accs[r], [d_idxs[u]],
                                       jnp.maximum(curs[u][r], valss[u][r]),
                                       mask=wons[u])
            all_won = wons[0]
            for u in range(1, UNROLL):
                all_won = jnp.logical_and(all_won, wons[u])

            @pl.when(jnp.logical_not(jnp.all(all_won)))
            def _():
                for u in range(UNROLL):
                    retry_scatter_max(d_idxs[u], valss[u],
                                      jnp.logical_not(wons[u]))

            return carry

        lax.fori_loop(0, VPC // UNROLL, vec_body, 0)

    issue(0, src_bufs.at[0], dst_bufs.at[0], sem0)

    def pair_body(k, carry):
        ci0 = 2 * k
        drain(src_bufs.at[0], dst_bufs.at[0], sem0)
        issue(ci0 + 1, src_bufs.at[1], dst_bufs.at[1], sem1)
        compute_chunk(0)
        drain(src_bufs.at[1], dst_bufs.at[1], sem1)
        issue(ci0 + 2, src_bufs.at[0], dst_bufs.at[0], sem0)
        compute_chunk(1)
        return carry

    lax.fori_loop(0, NCHUNK // 2, pair_body, 0)
    drain(src_bufs.at[0], dst_bufs.at[0], sem0)
    compute_chunk(0)

    def act_body(j, carry):
        for r in range(R):
            v = accs[r][pl.ds(j * L, L)]
            accs[r][pl.ds(j * L, L)] = jnp.where(v >= 0, v, NEG * v)
        return carry

    lax.fori_loop(0, N // L, act_body, 0)
    for r in range(R):
        pltpu.sync_copy(accs[r], out_hbm.at[wid, r])


def kernel(x, pos, edge_index):
    x_part = x.T.reshape(NW, 4, N)
    pos_part = jnp.concatenate(
        [pos.T, jnp.zeros((NW - 3, N), jnp.float32)], axis=0
    ).reshape(NW, 1, N)
    f_pad = jnp.concatenate([x_part, pos_part], axis=1)

    mesh = plsc.VectorSubcoreMesh(core_axis_name="c", subcore_axis_name="s",
                                  num_cores=2, num_subcores=16)
    out_pad = pl.kernel(
        _sc_body,
        out_type=jax.ShapeDtypeStruct((NW, R, N), jnp.float32),
        mesh=mesh,
        compiler_params=pltpu.CompilerParams(use_tc_tiling_on_sc=False,
                                             needs_layout_passes=False),
        scratch_types=[
            pltpu.VMEM((R, N), jnp.float32),
            pltpu.VMEM((N,), jnp.float32),
            pltpu.VMEM((N,), jnp.float32),
            pltpu.VMEM((N,), jnp.float32),
            pltpu.VMEM((N,), jnp.float32),
            pltpu.VMEM((N,), jnp.float32),
            pltpu.VMEM((N,), jnp.int32),
            pltpu.VMEM((2, CHUNK), jnp.int32),
            pltpu.VMEM((2, CHUNK), jnp.int32),
            pltpu.SemaphoreType.DMA,
            pltpu.SemaphoreType.DMA,
        ],
    )(f_pad, edge_index)

    out_rows = out_pad.reshape(NW * R, N)
    row_map = [5 * (c // 4) + c % 4 for c in range(D)] + [5 * p + 4
                                                         for p in range(3)]
    return out_rows[jnp.array(row_map), :].T


# speedup vs baseline: 2.1288x; 1.0042x over previous
import jax
import jax.numpy as jnp
from jax import lax
from jax.experimental import pallas as pl
from jax.experimental.pallas import tpu as pltpu
from jax.experimental.pallas import tpu_sc as plsc

N = 10000
D = 128
E = 320000
NEG = 0.01
L = 16
NW = 32
R = 5
CHUNK = 2560
NCHUNK = E // CHUNK
VPC = CHUNK // L
UNROLL = 4


def _sc_body(f_hbm, edge_hbm, out_hbm, f_buf, acc0, acc1, acc2,
             acc3, acc4, probe, src_bufs, dst_bufs, sem0, sem1):
    nc = 2
    wid = lax.axis_index("s") * nc + lax.axis_index("c")
    accs = [acc0, acc1, acc2, acc3, acc4]

    pltpu.sync_copy(f_hbm.at[wid], f_buf)
    for r in range(R - 1):
        pltpu.sync_copy(f_hbm.at[wid, r], accs[r])
    zeros = jnp.zeros((L,), jnp.float32)

    def zero_row4(j, carry):
        acc4[pl.ds(j * L, L)] = zeros
        return carry

    lax.fori_loop(0, N // L, zero_row4, 0)

    lane = lax.iota(jnp.int32, L)
    row_ids = [jnp.full((L,), r, jnp.int32) for r in range(R)]
    tags = [lane + u * L for u in range(UNROLL)]

    def issue(ci, sbuf, dbuf, sem):
        e0 = ci * CHUNK
        pltpu.async_copy(edge_hbm.at[0, pl.ds(e0, CHUNK)], sbuf, sem)
        pltpu.async_copy(edge_hbm.at[1, pl.ds(e0, CHUNK)], dbuf, sem)

    def drain(sbuf, dbuf, sem):
        pltpu.make_async_copy(edge_hbm.at[0, pl.ds(0, CHUNK)], sbuf,
                              sem).wait()
        pltpu.make_async_copy(edge_hbm.at[1, pl.ds(0, CHUNK)], dbuf,
                              sem).wait()

    def retry_scatter_max(d_idx, vals, active0):
        def w_cond(active):
            return jnp.any(active)

        def w_body(active):
            plsc.store_scatter(probe, [d_idx], lane, mask=active)
            back = plsc.load_gather(probe, [d_idx])
            won = jnp.logical_and(back == lane, active)
            for r in range(R):
                cur = plsc.load_gather(accs[r], [d_idx])
                plsc.store_scatter(accs[r], [d_idx],
                                   jnp.maximum(cur, vals[r]), mask=won)
            return jnp.logical_and(active, jnp.logical_not(won))

        lax.while_loop(w_cond, w_body, active0)

    def compute_chunk(p):
        def vec_body(vi, carry):
            d_idxs, valss = [], []
            for u in range(UNROLL):
                v = vi * UNROLL + u
                s_idx = src_bufs[p, pl.ds(v * L, L)]
                d_idx = dst_bufs[p, pl.ds(v * L, L)]
                vals = [plsc.load_gather(f_buf, [row_ids[r], s_idx])
                        for r in range(R - 1)]
                vals.append(
                    plsc.load_gather(f_buf, [row_ids[R - 1], s_idx])
                    - plsc.load_gather(f_buf, [row_ids[R - 1], d_idx]))
                d_idxs.append(d_idx)
                valss.append(vals)
            for u in range(UNROLL):
                plsc.store_scatter(probe, [d_idxs[u]], tags[u])
            backs = [plsc.load_gather(probe, [d_idxs[u]])
                     for u in range(UNROLL)]
            wons = [backs[u] == tags[u] for u in range(UNROLL)]
            curs = [[plsc.load_gather(accs[r], [d_idxs[u]])
                     for r in range(R)] for u in range(UNROLL)]
            for u in range(UNROLL):
                for r in range(R):
                    plsc.store_scatter(accs[r], [d_idxs[u]],
                                       jnp.maximum(curs[u][r], valss[u][r]),
                                       mask=wons[u])
            all_won = wons[0]
            for u in range(1, UNROLL):
                all_won = jnp.logical_and(all_won, wons[u])
            n_lost = plsc.all_reduce_population_count(
                jnp.logical_not(all_won))

            @pl.when(n_lost[0] > 0)
            def _():
                for u in range(UNROLL):
                    retry_scatter_max(d_idxs[u], valss[u],
                                      jnp.logical_not(wons[u]))

            return carry

        lax.fori_loop(0, VPC // UNROLL, vec_body, 0)

    issue(0, src_bufs.at[0], dst_bufs.at[0], sem0)

    def pair_body(k, carry):
        ci0 = 2 * k
        drain(src_bufs.at[0], dst_bufs.at[0], sem0)
        issue(ci0 + 1, src_bufs.at[1], dst_bufs.at[1], sem1)
        compute_chunk(0)
        drain(src_bufs.at[1], dst_bufs.at[1], sem1)
        issue(ci0 + 2, src_bufs.at[0], dst_bufs.at[0], sem0)
        compute_chunk(1)
        return carry

    lax.fori_loop(0, NCHUNK // 2, pair_body, 0)
    drain(src_bufs.at[0], dst_bufs.at[0], sem0)
    compute_chunk(0)

    def act_body(j, carry):
        for r in range(R):
            v = accs[r][pl.ds(j * L, L)]
            accs[r][pl.ds(j * L, L)] = jnp.where(v >= 0, v, NEG * v)
        return carry

    lax.fori_loop(0, N // L, act_body, 0)
    for r in range(R):
        pltpu.sync_copy(accs[r], out_hbm.at[wid, r])


def kernel(x, pos, edge_index):
    x_part = x.T.reshape(NW, 4, N)
    pos_part = jnp.concatenate(
        [pos.T, jnp.zeros((NW - 3, N), jnp.float32)], axis=0
    ).reshape(NW, 1, N)
    f_pad = jnp.concatenate([x_part, pos_part], axis=1)

    mesh = plsc.VectorSubcoreMesh(core_axis_name="c", subcore_axis_name="s",
                                  num_cores=2, num_subcores=16)
    out_pad = pl.kernel(
        _sc_body,
        out_type=jax.ShapeDtypeStruct((NW, R, N), jnp.float32),
        mesh=mesh,
        compiler_params=pltpu.CompilerParams(use_tc_tiling_on_sc=False,
                                             needs_layout_passes=False),
        scratch_types=[
            pltpu.VMEM((R, N), jnp.float32),
            pltpu.VMEM((N,), jnp.float32),
            pltpu.VMEM((N,), jnp.float32),
            pltpu.VMEM((N,), jnp.float32),
            pltpu.VMEM((N,), jnp.float32),
            pltpu.VMEM((N,), jnp.float32),
            pltpu.VMEM((N,), jnp.int32),
            pltpu.VMEM((2, CHUNK), jnp.int32),
            pltpu.VMEM((2, CHUNK), jnp.int32),
            pltpu.SemaphoreType.DMA,
            pltpu.SemaphoreType.DMA,
        ],
    )(f_pad, edge_index)

    out_rows = out_pad.reshape(NW * R, N)
    row_map = [5 * (c // 4) + c % 4 for c in range(D)] + [5 * p + 4
                                                         for p in range(3)]
    return out_rows[jnp.array(row_map), :].T
